# baseline (speedup 1.0000x reference)
import functools

import jax
import jax.numpy as jnp
from jax import lax
from jax.experimental import pallas as pl
from jax.experimental.pallas import tpu as pltpu
from jax.experimental.pallas import tpu_sc as plsc

_PERIOD = 4
_START = 0
_NC = 2
_NS = 16
_NW = _NC * _NS


@functools.partial(jax.jit, static_argnames=("tot_rows", "d", "off"))
def _sc_decimate(x_r4, tot_rows, d, off):
    mesh = plsc.VectorSubcoreMesh(
        core_axis_name="c", subcore_axis_name="s",
        num_cores=_NC, num_subcores=_NS,
    )
    rows_per_w = tot_rows // _NW

    @functools.partial(
        pl.kernel,
        out_type=jax.ShapeDtypeStruct((tot_rows, d), jnp.float32),
        mesh=mesh,
    )
    def run(x_hbm, out_hbm):
        wid = lax.axis_index("s") * _NC + lax.axis_index("c")
        base = wid * rows_per_w
        pltpu.sync_copy(
            x_hbm.at[pl.ds(base, rows_per_w), off],
            out_hbm.at[pl.ds(base, rows_per_w)],
        )

    return run(x_r4)


def kernel(x, dim):
    b, n, d = x.shape
    off = jnp.asarray(dim, dtype=jnp.int32) - 1 + _START
    r_out = (n - _START + _PERIOD - 1) // _PERIOD
    tot_rows = b * r_out
    x_r4 = x.reshape(tot_rows, _PERIOD, d)
    branches = [
        functools.partial(_sc_decimate, tot_rows=tot_rows, d=d, off=p)
        for p in range(_PERIOD)
    ]
    out_flat = lax.switch(off, branches, x_r4)
    return out_flat.reshape(b, r_out, d)
